# baseline (speedup 1.0000x reference)
import jax
import jax.numpy as jnp
from jax.experimental import pallas as pl

_HEADS = 12
_HEAD_DIM = 64
_SEQ = 2048
_NHASH = 2
_NBUCK = 64


def _bucket_kernel(hid_ref, rot_ref, bc_ref, br_ref):
    subiota = jax.lax.broadcasted_iota(jnp.int32, (_NBUCK, _SEQ), 0)
    vcol = jax.lax.broadcasted_iota(
        jnp.int32, (_NBUCK, 1), 0).astype(jnp.bfloat16)
    for a in range(2):
        v = hid_ref[:, _HEAD_DIM * a:_HEAD_DIM * (a + 1)]
        r = rot_ref[a]
        rott = jax.lax.dot_general(
            r, v, (((0,), (1,)), ((), ())),
            preferred_element_type=jnp.float32)
        for k in range(_NHASH):
            x = rott[32 * k:32 * k + 32, :]
            full = jnp.concatenate([x, -x], axis=0)
            mx = jnp.max(full, axis=0, keepdims=True)
            brow = jnp.min(jnp.where(full == mx, subiota, _NBUCK),
                           axis=0, keepdims=True)
            onehot = (subiota == brow).astype(jnp.bfloat16)
            bcol = jax.lax.dot_general(
                onehot, vcol, (((0,), (0,)), ((), ())),
                preferred_element_type=jnp.float32)
            br_ref[a, k:k + 1, :] = brow.astype(jnp.int16)
            bc_ref[a, :, k:k + 1] = bcol.astype(jnp.int16)


def _cmp_kernel(bc_ref, br_ref, out_ref):
    bc0 = bc_ref[0, :, 0:1]
    bc1 = bc_ref[0, :, 1:2]
    br0 = br_ref[0, 0:1, :]
    br1 = br_ref[0, 1:2, :]
    out_ref[0, 0] = ((bc0 == br0) | (bc1 == br1)).astype(jnp.int8)


def kernel(hidden_states, rotations):
    hid2d = hidden_states.reshape(_SEQ, _HEADS * _HEAD_DIM)
    rot3d = rotations.reshape(_HEADS, _HEAD_DIM, _NHASH * (_NBUCK // 2))
    bc, br = pl.pallas_call(
        _bucket_kernel,
        grid=(_HEADS // 2,),
        in_specs=[
            pl.BlockSpec((_SEQ, 2 * _HEAD_DIM), lambda g: (0, g)),
            pl.BlockSpec((2, _HEAD_DIM, _NBUCK), lambda g: (g, 0, 0)),
        ],
        out_specs=[
            pl.BlockSpec((2, _SEQ, _NHASH), lambda g: (g, 0, 0)),
            pl.BlockSpec((2, _NHASH, _SEQ), lambda g: (g, 0, 0)),
        ],
        out_shape=[
            jax.ShapeDtypeStruct((_HEADS, _SEQ, _NHASH), jnp.int16),
            jax.ShapeDtypeStruct((_HEADS, _NHASH, _SEQ), jnp.int16),
        ],
    )(hid2d, rot3d)
    out = pl.pallas_call(
        _cmp_kernel,
        grid=(_HEADS,),
        in_specs=[
            pl.BlockSpec((1, _SEQ, _NHASH), lambda h: (h, 0, 0)),
            pl.BlockSpec((1, _NHASH, _SEQ), lambda h: (h, 0, 0)),
        ],
        out_specs=pl.BlockSpec((1, 1, _SEQ, _SEQ), lambda h: (0, h, 0, 0)),
        out_shape=jax.ShapeDtypeStruct((1, _HEADS, _SEQ, _SEQ), jnp.int8),
    )(bc, br)
    return out.astype(jnp.bool_)
